# baseline (speedup 1.0000x reference)
import functools

import jax
import jax.numpy as jnp
from jax.experimental import pallas as pl
from jax.experimental.pallas import tpu as pltpu

_BATCH = 1024
_D = 16
_BLOCK_N = 2048


def _fused_kernel(user_ref, w_ref, b_ref, gamma_ref, beta_ref, item_ref,
                  out_ref):
    h = jnp.dot(user_ref[:], w_ref[:].T,
                preferred_element_type=jnp.float32) + b_ref[:]
    mu = jnp.mean(h, axis=-1, keepdims=True)
    d = h - mu
    var = jnp.mean(d * d, axis=-1, keepdims=True)
    h = d * jax.lax.rsqrt(var + 1e-5) * gamma_ref[:] + beta_ref[:]
    out_ref[:] = jax.lax.dot_general(
        h, item_ref[:], (((1,), (1,)), ((), ())),
        preferred_element_type=jnp.float32)


@jax.jit
def kernel(user_emb, item_emb, W, b, gamma, beta):
    n_items = item_emb.shape[0]
    grid = (pl.cdiv(n_items, _BLOCK_N),)
    b2 = b.reshape(1, _D)
    gamma2 = gamma.reshape(1, _D)
    beta2 = beta.reshape(1, _D)
    return pl.pallas_call(
        _fused_kernel,
        grid=grid,
        in_specs=[
            pl.BlockSpec((_BATCH, _D), lambda i: (0, 0)),
            pl.BlockSpec((_D, _D), lambda i: (0, 0)),
            pl.BlockSpec((1, _D), lambda i: (0, 0)),
            pl.BlockSpec((1, _D), lambda i: (0, 0)),
            pl.BlockSpec((1, _D), lambda i: (0, 0)),
            pl.BlockSpec((_BLOCK_N, _D), lambda i: (i, 0)),
        ],
        out_specs=pl.BlockSpec((_BATCH, _BLOCK_N), lambda i: (0, i)),
        out_shape=jax.ShapeDtypeStruct((_BATCH, n_items), jnp.float32),
        compiler_params=pltpu.CompilerParams(
            dimension_semantics=("parallel",)),
    )(user_emb, W, b2, gamma2, beta2, item_emb)


# speedup vs baseline: 1.0259x; 1.0259x over previous
import functools

import jax
import jax.numpy as jnp
from jax.experimental import pallas as pl
from jax.experimental.pallas import tpu as pltpu

_BATCH = 1024
_D = 16
_BLOCK_N = 4096


def _fused_kernel(user_ref, w_ref, b_ref, gamma_ref, beta_ref, item_ref,
                  out_ref):
    h = jnp.dot(user_ref[:], w_ref[:].T,
                preferred_element_type=jnp.float32) + b_ref[:]
    mu = jnp.mean(h, axis=-1, keepdims=True)
    d = h - mu
    var = jnp.mean(d * d, axis=-1, keepdims=True)
    h = d * jax.lax.rsqrt(var + 1e-5) * gamma_ref[:] + beta_ref[:]
    out_ref[:] = jax.lax.dot_general(
        h, item_ref[:], (((1,), (1,)), ((), ())),
        preferred_element_type=jnp.float32)


@jax.jit
def kernel(user_emb, item_emb, W, b, gamma, beta):
    n_items = item_emb.shape[0]
    grid = (pl.cdiv(n_items, _BLOCK_N),)
    b2 = b.reshape(1, _D)
    gamma2 = gamma.reshape(1, _D)
    beta2 = beta.reshape(1, _D)
    return pl.pallas_call(
        _fused_kernel,
        grid=grid,
        in_specs=[
            pl.BlockSpec((_BATCH, _D), lambda i: (0, 0)),
            pl.BlockSpec((_D, _D), lambda i: (0, 0)),
            pl.BlockSpec((1, _D), lambda i: (0, 0)),
            pl.BlockSpec((1, _D), lambda i: (0, 0)),
            pl.BlockSpec((1, _D), lambda i: (0, 0)),
            pl.BlockSpec((_BLOCK_N, _D), lambda i: (i, 0)),
        ],
        out_specs=pl.BlockSpec((_BATCH, _BLOCK_N), lambda i: (0, i)),
        out_shape=jax.ShapeDtypeStruct((_BATCH, n_items), jnp.float32),
        compiler_params=pltpu.CompilerParams(
            dimension_semantics=("parallel",)),
    )(user_emb, W, b2, gamma2, beta2, item_emb)
